# baseline (speedup 1.0000x reference)
import functools

import jax
import jax.numpy as jnp
from jax import lax
from jax.experimental import pallas as pl
from jax.experimental.pallas import tpu as pltpu
from jax.experimental.pallas import tpu_sc as plsc

_NC = 2
_NS = 16
_K = 100
_NB = 2


def _sc_mesh():
    return plsc.VectorSubcoreMesh(
        core_axis_name="c", subcore_axis_name="s",
        num_cores=_NC, num_subcores=_NS)


def _make_degree_kernel(N, E):
    nworkers = _NC * _NS
    eper = E // nworkers
    nch = eper // _K
    rows = N // _NS

    @functools.partial(
        pl.kernel,
        out_type=jax.ShapeDtypeStruct((_NC, N, 16), jnp.float32),
        mesh=_sc_mesh(),
        compiler_params=pltpu.CompilerParams(use_tc_tiling_on_sc=False),
        scratch_types=[
            pltpu.VMEM((nch, _K), jnp.int32),
            pltpu.VMEM((_K, 16), jnp.float32),
            pltpu.VMEM_SHARED((N, 16), jnp.float32),
        ],
    )
    def deg_kernel(dst_hbm, ones_hbm, zeros_hbm, out_hbm, dst_v, ones_v, acc_sh):
        c = lax.axis_index("c")
        s = lax.axis_index("s")
        wid = c * _NS + s
        pltpu.sync_copy(dst_hbm.at[wid], dst_v)
        pltpu.sync_copy(ones_hbm, ones_v)
        pltpu.sync_copy(zeros_hbm.at[pl.ds(s * rows, rows)],
                        acc_sh.at[pl.ds(s * rows, rows)])
        plsc.subcore_barrier()

        def body(i, carry):
            pltpu.sync_copy(ones_v, acc_sh.at[dst_v.at[i]], add=True)
            return carry

        lax.fori_loop(0, nch, body, 0)
        plsc.subcore_barrier()
        pltpu.sync_copy(acc_sh.at[pl.ds(s * rows, rows)],
                        out_hbm.at[c].at[pl.ds(s * rows, rows)])

    return deg_kernel


def _make_scatter_kernel(N, E, F):
    nworkers = _NC * _NS
    eper = E // nworkers
    nch = eper // _K
    rows = N // _NS

    @functools.partial(
        pl.kernel,
        out_type=jax.ShapeDtypeStruct((_NC, N, F), jnp.float32),
        mesh=_sc_mesh(),
        compiler_params=pltpu.CompilerParams(use_tc_tiling_on_sc=False),
        scratch_types=[
            pltpu.VMEM((nch, _K), jnp.int32),
            pltpu.VMEM((nch, _K), jnp.int32),
            [pltpu.VMEM((_K, F), jnp.float32)] * _NB,
            [pltpu.SemaphoreType.DMA] * _NB,
            [pltpu.SemaphoreType.DMA] * _NB,
            pltpu.VMEM_SHARED((N, F), jnp.float32),
        ],
    )
    def scatter_kernel(y_hbm, src_hbm, dst_hbm, zeros_hbm, out_hbm,
                       src_v, dst_v, rows_t, gsem, ssem, acc_sh):
        c = lax.axis_index("c")
        s = lax.axis_index("s")
        wid = c * _NS + s
        pltpu.sync_copy(src_hbm.at[wid], src_v)
        pltpu.sync_copy(dst_hbm.at[wid], dst_v)
        pltpu.sync_copy(zeros_hbm.at[pl.ds(s * rows, rows)],
                        acc_sh.at[pl.ds(s * rows, rows)])
        plsc.subcore_barrier()

        def gather(i, t):
            pltpu.async_copy(y_hbm.at[src_v.at[i]], rows_t[t], gsem[t])

        def gwait(i, t):
            pltpu.make_async_copy(y_hbm.at[src_v.at[i]], rows_t[t],
                                  gsem[t]).wait()

        def scatter(i, t):
            pltpu.async_copy(rows_t[t], acc_sh.at[dst_v.at[i]],
                             ssem[t], add=True)

        def swait(i, t):
            pltpu.make_async_copy(rows_t[t], acc_sh.at[dst_v.at[i]],
                                  ssem[t]).wait()

        for t in range(_NB):
            gather(t, t)

        def body(j, carry):
            base = j * _NB
            for t in range(_NB):
                gwait(base + t, t)
                scatter(base + t, t)
            for t in range(_NB):
                swait(base + t, t)
                gather(base + _NB + t, t)
            return carry

        lax.fori_loop(0, nch // _NB - 1, body, 0)
        last = nch - _NB
        for t in range(_NB):
            gwait(last + t, t)
            scatter(last + t, t)
        for t in range(_NB):
            swait(last + t, t)
        plsc.subcore_barrier()
        pltpu.sync_copy(acc_sh.at[pl.ds(s * rows, rows)],
                        out_hbm.at[c].at[pl.ds(s * rows, rows)])

    return scatter_kernel


def _tc_first(x, W, deg_parts):
    N = x.shape[0]
    H = W.shape[1]

    def body(x_ref, w_ref, dp_ref, y_ref, dinv_ref):
        deg = dp_ref[0, :, 0:1] + dp_ref[1, :, 0:1] + 1.0
        dinv = lax.rsqrt(deg)
        xw = jnp.dot(x_ref[...], w_ref[...], preferred_element_type=jnp.float32)
        y_ref[...] = dinv * xw
        dinv_ref[...] = dinv

    return pl.pallas_call(
        body,
        out_shape=[jax.ShapeDtypeStruct((N, H), jnp.float32),
                   jax.ShapeDtypeStruct((N, 1), jnp.float32)],
    )(x, W, deg_parts)


def _tc_mid(zp, y, dinv, b, W):
    N = y.shape[0]
    H2 = W.shape[1]

    def body(zp_ref, y_ref, dinv_ref, b_ref, w_ref, out_ref):
        z = zp_ref[0] + zp_ref[1] + y_ref[...]
        h = jnp.maximum(dinv_ref[...] * z + b_ref[...], 0.0)
        out_ref[...] = dinv_ref[...] * jnp.dot(
            h, w_ref[...], preferred_element_type=jnp.float32)

    return pl.pallas_call(
        body,
        out_shape=jax.ShapeDtypeStruct((N, H2), jnp.float32),
    )(zp, y, dinv, b, W)


def _tc_last(zp, y, dinv, b):
    N, F = y.shape

    def body(zp_ref, y_ref, dinv_ref, b_ref, out_ref):
        z = zp_ref[0] + zp_ref[1] + y_ref[...]
        out_ref[...] = dinv_ref[...] * z + b_ref[...]

    return pl.pallas_call(
        body,
        out_shape=jax.ShapeDtypeStruct((N, F), jnp.float32),
    )(zp, y, dinv, b)


def kernel(x, edge_index, W1, b1, W2, b2, W3, b3):
    N = x.shape[0]
    E = edge_index.shape[1]
    H = W1.shape[1]
    C = W3.shape[1]
    F3 = 48

    nworkers = _NC * _NS
    nch = E // nworkers // _K
    src = edge_index[0].reshape(nworkers, nch, _K)
    dst = edge_index[1].reshape(nworkers, nch, _K)
    W3p = jnp.pad(W3, ((0, 0), (0, F3 - C)))
    b3p = jnp.pad(b3, (0, F3 - C))

    ones16 = jnp.ones((_K, 16), jnp.float32)
    zeros16 = jnp.zeros((N, 16), jnp.float32)
    zerosH = jnp.zeros((N, H), jnp.float32)
    zerosF3 = jnp.zeros((N, F3), jnp.float32)

    deg_parts = _make_degree_kernel(N, E)(dst, ones16, zeros16)
    y1, dinv = _tc_first(x, W1, deg_parts)
    zp1 = _make_scatter_kernel(N, E, H)(y1, src, dst, zerosH)
    y2 = _tc_mid(zp1, y1, dinv, b1.reshape(1, H), W2)
    zp2 = _make_scatter_kernel(N, E, H)(y2, src, dst, zerosH)
    y3 = _tc_mid(zp2, y2, dinv, b2.reshape(1, H), W3p)
    zp3 = _make_scatter_kernel(N, E, F3)(y3, src, dst, zerosF3)
    out = _tc_last(zp3, y3, dinv, b3p.reshape(1, F3))
    return out[:, :C]


# speedup vs baseline: 1.1957x; 1.1957x over previous
import functools

import jax
import jax.numpy as jnp
from jax import lax
from jax.experimental import pallas as pl
from jax.experimental.pallas import tpu as pltpu
from jax.experimental.pallas import tpu_sc as plsc

_NC = 2
_NS = 16
_K = 100


def _sc_mesh():
    return plsc.VectorSubcoreMesh(
        core_axis_name="c", subcore_axis_name="s",
        num_cores=_NC, num_subcores=_NS)


def _make_degree_kernel(N, E):
    nworkers = _NC * _NS
    eper = E // nworkers
    nch = eper // _K
    rows = N // _NS

    @functools.partial(
        pl.kernel,
        out_type=jax.ShapeDtypeStruct((_NC, N, 16), jnp.float32),
        mesh=_sc_mesh(),
        compiler_params=pltpu.CompilerParams(use_tc_tiling_on_sc=False),
        scratch_types=[
            pltpu.VMEM((nch, _K), jnp.int32),
            pltpu.VMEM((_K, 16), jnp.float32),
            pltpu.VMEM_SHARED((N, 16), jnp.float32),
        ],
    )
    def deg_kernel(dst_hbm, ones_hbm, zeros_hbm, out_hbm, dst_v, ones_v, acc_sh):
        c = lax.axis_index("c")
        s = lax.axis_index("s")
        wid = c * _NS + s
        pltpu.sync_copy(dst_hbm.at[wid], dst_v)
        pltpu.sync_copy(ones_hbm, ones_v)
        pltpu.sync_copy(zeros_hbm.at[pl.ds(s * rows, rows)],
                        acc_sh.at[pl.ds(s * rows, rows)])
        plsc.subcore_barrier()

        def body(i, carry):
            pltpu.sync_copy(ones_v, acc_sh.at[dst_v.at[i]], add=True)
            return carry

        lax.fori_loop(0, nch, body, 0)
        plsc.subcore_barrier()
        pltpu.sync_copy(acc_sh.at[pl.ds(s * rows, rows)],
                        out_hbm.at[c].at[pl.ds(s * rows, rows)])

    return deg_kernel


def _make_scatter_kernel(N, E, F):
    nworkers = _NC * _NS
    eper = E // nworkers
    nch = eper // _K
    rows = N // _NS

    @functools.partial(
        pl.kernel,
        out_type=jax.ShapeDtypeStruct((_NC, N, F), jnp.float32),
        mesh=_sc_mesh(),
        compiler_params=pltpu.CompilerParams(use_tc_tiling_on_sc=False),
        scratch_types=[
            pltpu.VMEM((nch, _K), jnp.int32),
            pltpu.VMEM((nch, _K), jnp.int32),
            pltpu.VMEM((_K, F), jnp.float32),
            pltpu.VMEM((_K, F), jnp.float32),
            pltpu.VMEM_SHARED((N, F), jnp.float32),
            pltpu.SemaphoreType.DMA,
            pltpu.SemaphoreType.DMA,
        ],
    )
    def scatter_kernel(y_hbm, src_hbm, dst_hbm, zeros_hbm, out_hbm,
                       src_v, dst_v, rows_a, rows_b, acc_sh, sem_a, sem_b):
        c = lax.axis_index("c")
        s = lax.axis_index("s")
        wid = c * _NS + s
        pltpu.sync_copy(src_hbm.at[wid], src_v)
        pltpu.sync_copy(dst_hbm.at[wid], dst_v)
        pltpu.sync_copy(zeros_hbm.at[pl.ds(s * rows, rows)],
                        acc_sh.at[pl.ds(s * rows, rows)])
        plsc.subcore_barrier()

        def gather(i, buf, sem):
            return pltpu.async_copy(y_hbm.at[src_v.at[i]], buf, sem)

        def gwait(i, buf, sem):
            pltpu.make_async_copy(y_hbm.at[src_v.at[i]], buf, sem).wait()

        def scatter(i, buf):
            pltpu.sync_copy(buf, acc_sh.at[dst_v.at[i]], add=True)

        gather(0, rows_a, sem_a)

        def body(j, carry):
            i0 = 2 * j
            i1 = 2 * j + 1
            gather(i1, rows_b, sem_b)
            gwait(i0, rows_a, sem_a)
            scatter(i0, rows_a)

            @pl.when(i0 + 2 < nch)
            def _():
                gather(i0 + 2, rows_a, sem_a)

            gwait(i1, rows_b, sem_b)
            scatter(i1, rows_b)
            return carry

        lax.fori_loop(0, nch // 2, body, 0)
        plsc.subcore_barrier()
        pltpu.sync_copy(acc_sh.at[pl.ds(s * rows, rows)],
                        out_hbm.at[c].at[pl.ds(s * rows, rows)])

    return scatter_kernel


def _tc_first(x, W, deg_parts):
    N = x.shape[0]
    H = W.shape[1]

    def body(x_ref, w_ref, dp_ref, y_ref, dinv_ref):
        deg = dp_ref[0, :, 0:1] + dp_ref[1, :, 0:1] + 1.0
        dinv = lax.rsqrt(deg)
        xw = jnp.dot(x_ref[...], w_ref[...], preferred_element_type=jnp.float32)
        y_ref[...] = dinv * xw
        dinv_ref[...] = dinv

    return pl.pallas_call(
        body,
        out_shape=[jax.ShapeDtypeStruct((N, H), jnp.float32),
                   jax.ShapeDtypeStruct((N, 1), jnp.float32)],
    )(x, W, deg_parts)


def _tc_mid(zp, y, dinv, b, W):
    N = y.shape[0]
    H2 = W.shape[1]

    def body(zp_ref, y_ref, dinv_ref, b_ref, w_ref, out_ref):
        z = zp_ref[0] + zp_ref[1] + y_ref[...]
        h = jnp.maximum(dinv_ref[...] * z + b_ref[...], 0.0)
        out_ref[...] = dinv_ref[...] * jnp.dot(
            h, w_ref[...], preferred_element_type=jnp.float32)

    return pl.pallas_call(
        body,
        out_shape=jax.ShapeDtypeStruct((N, H2), jnp.float32),
    )(zp, y, dinv, b, W)


def _tc_last(zp, y, dinv, b):
    N, F = y.shape

    def body(zp_ref, y_ref, dinv_ref, b_ref, out_ref):
        z = zp_ref[0] + zp_ref[1] + y_ref[...]
        out_ref[...] = dinv_ref[...] * z + b_ref[...]

    return pl.pallas_call(
        body,
        out_shape=jax.ShapeDtypeStruct((N, F), jnp.float32),
    )(zp, y, dinv, b)


def kernel(x, edge_index, W1, b1, W2, b2, W3, b3):
    N = x.shape[0]
    E = edge_index.shape[1]
    H = W1.shape[1]
    C = W3.shape[1]
    F3 = 48

    nworkers = _NC * _NS
    nch = E // nworkers // _K
    src = edge_index[0].reshape(nworkers, nch, _K)
    dst = edge_index[1].reshape(nworkers, nch, _K)
    W3p = jnp.pad(W3, ((0, 0), (0, F3 - C)))
    b3p = jnp.pad(b3, (0, F3 - C))

    ones16 = jnp.ones((_K, 16), jnp.float32)
    zeros16 = jnp.zeros((N, 16), jnp.float32)
    zerosH = jnp.zeros((N, H), jnp.float32)
    zerosF3 = jnp.zeros((N, F3), jnp.float32)

    deg_parts = _make_degree_kernel(N, E)(dst, ones16, zeros16)
    y1, dinv = _tc_first(x, W1, deg_parts)
    zp1 = _make_scatter_kernel(N, E, H)(y1, src, dst, zerosH)
    y2 = _tc_mid(zp1, y1, dinv, b1.reshape(1, H), W2)
    zp2 = _make_scatter_kernel(N, E, H)(y2, src, dst, zerosH)
    y3 = _tc_mid(zp2, y2, dinv, b2.reshape(1, H), W3p)
    zp3 = _make_scatter_kernel(N, E, F3)(y3, src, dst, zerosF3)
    out = _tc_last(zp3, y3, dinv, b3p.reshape(1, F3))
    return out[:, :C]


# speedup vs baseline: 1.2065x; 1.0090x over previous
import functools

import jax
import jax.numpy as jnp
from jax import lax
from jax.experimental import pallas as pl
from jax.experimental.pallas import tpu as pltpu
from jax.experimental.pallas import tpu_sc as plsc

_NC = 2
_NS = 16
_K = 100


def _sc_mesh():
    return plsc.VectorSubcoreMesh(
        core_axis_name="c", subcore_axis_name="s",
        num_cores=_NC, num_subcores=_NS)


def _make_degree_kernel(N, E):
    nworkers = _NC * _NS
    eper = E // nworkers
    nch = eper // _K
    rows = N // _NS

    @functools.partial(
        pl.kernel,
        out_type=jax.ShapeDtypeStruct((_NC, N, 16), jnp.float32),
        mesh=_sc_mesh(),
        compiler_params=pltpu.CompilerParams(use_tc_tiling_on_sc=False),
        scratch_types=[
            pltpu.VMEM((nch, _K), jnp.int32),
            pltpu.VMEM((_K, 16), jnp.float32),
            pltpu.VMEM_SHARED((N, 16), jnp.float32),
        ],
    )
    def deg_kernel(dst_hbm, ones_hbm, zeros_hbm, out_hbm, dst_v, ones_v, acc_sh):
        c = lax.axis_index("c")
        s = lax.axis_index("s")
        wid = c * _NS + s
        pltpu.sync_copy(dst_hbm.at[wid], dst_v)
        pltpu.sync_copy(ones_hbm, ones_v)
        pltpu.sync_copy(zeros_hbm.at[pl.ds(s * rows, rows)],
                        acc_sh.at[pl.ds(s * rows, rows)])
        plsc.subcore_barrier()

        def body(i, carry):
            pltpu.sync_copy(ones_v, acc_sh.at[dst_v.at[i]], add=True)
            return carry

        lax.fori_loop(0, nch, body, 0)
        plsc.subcore_barrier()
        pltpu.sync_copy(acc_sh.at[pl.ds(s * rows, rows)],
                        out_hbm.at[c].at[pl.ds(s * rows, rows)])

    return deg_kernel


def _make_scatter_kernel(N, E, F):
    nworkers = _NC * _NS
    eper = E // nworkers
    nch = eper // _K
    rows = N // _NS

    @functools.partial(
        pl.kernel,
        out_type=jax.ShapeDtypeStruct((_NC, N, F), jnp.float32),
        mesh=_sc_mesh(),
        compiler_params=pltpu.CompilerParams(use_tc_tiling_on_sc=False),
        scratch_types=[
            pltpu.VMEM((nch, _K), jnp.int32),
            pltpu.VMEM((nch, _K), jnp.int32),
            pltpu.VMEM((_K, F), jnp.float32),
            pltpu.VMEM((_K, F), jnp.float32),
            pltpu.VMEM_SHARED((N, F), jnp.float32),
            pltpu.SemaphoreType.DMA,
            pltpu.SemaphoreType.DMA,
        ],
    )
    def scatter_kernel(y_hbm, src_hbm, dst_hbm, zeros_hbm, out_hbm,
                       src_v, dst_v, rows_a, rows_b, acc_sh, sem_a, sem_b):
        c = lax.axis_index("c")
        s = lax.axis_index("s")
        wid = c * _NS + s
        pltpu.sync_copy(src_hbm.at[wid], src_v)
        pltpu.sync_copy(dst_hbm.at[wid], dst_v)

        def gather(i, buf, sem):
            return pltpu.async_copy(y_hbm.at[src_v.at[i]], buf, sem)

        def gwait(i, buf, sem):
            pltpu.make_async_copy(y_hbm.at[src_v.at[i]], buf, sem).wait()

        def scatter(i, buf):
            pltpu.sync_copy(buf, acc_sh.at[dst_v.at[i]], add=True)

        gather(0, rows_a, sem_a)
        pltpu.sync_copy(zeros_hbm.at[pl.ds(s * rows, rows)],
                        acc_sh.at[pl.ds(s * rows, rows)])
        plsc.subcore_barrier()


        def body(j, carry):
            i0 = 2 * j
            i1 = 2 * j + 1
            gather(i1, rows_b, sem_b)
            gwait(i0, rows_a, sem_a)
            scatter(i0, rows_a)

            @pl.when(i0 + 2 < nch)
            def _():
                gather(i0 + 2, rows_a, sem_a)

            gwait(i1, rows_b, sem_b)
            scatter(i1, rows_b)
            return carry

        lax.fori_loop(0, nch // 2, body, 0)
        plsc.subcore_barrier()
        pltpu.sync_copy(acc_sh.at[pl.ds(s * rows, rows)],
                        out_hbm.at[c].at[pl.ds(s * rows, rows)])

    return scatter_kernel


def _tc_first(x, W, deg_parts):
    N = x.shape[0]
    H = W.shape[1]

    def body(x_ref, w_ref, dp_ref, y_ref, dinv_ref):
        deg = dp_ref[0, :, 0:1] + dp_ref[1, :, 0:1] + 1.0
        dinv = lax.rsqrt(deg)
        xw = jnp.dot(x_ref[...], w_ref[...], preferred_element_type=jnp.float32)
        y_ref[...] = dinv * xw
        dinv_ref[...] = dinv

    return pl.pallas_call(
        body,
        out_shape=[jax.ShapeDtypeStruct((N, H), jnp.float32),
                   jax.ShapeDtypeStruct((N, 1), jnp.float32)],
    )(x, W, deg_parts)


def _tc_mid(zp, y, dinv, b, W):
    N = y.shape[0]
    H2 = W.shape[1]

    def body(zp_ref, y_ref, dinv_ref, b_ref, w_ref, out_ref):
        z = zp_ref[0] + zp_ref[1] + y_ref[...]
        h = jnp.maximum(dinv_ref[...] * z + b_ref[...], 0.0)
        out_ref[...] = dinv_ref[...] * jnp.dot(
            h, w_ref[...], preferred_element_type=jnp.float32)

    return pl.pallas_call(
        body,
        out_shape=jax.ShapeDtypeStruct((N, H2), jnp.float32),
    )(zp, y, dinv, b, W)


def _tc_last(zp, y, dinv, b):
    N, F = y.shape

    def body(zp_ref, y_ref, dinv_ref, b_ref, out_ref):
        z = zp_ref[0] + zp_ref[1] + y_ref[...]
        out_ref[...] = dinv_ref[...] * z + b_ref[...]

    return pl.pallas_call(
        body,
        out_shape=jax.ShapeDtypeStruct((N, F), jnp.float32),
    )(zp, y, dinv, b)


def kernel(x, edge_index, W1, b1, W2, b2, W3, b3):
    N = x.shape[0]
    E = edge_index.shape[1]
    H = W1.shape[1]
    C = W3.shape[1]
    F3 = 48

    nworkers = _NC * _NS
    nch = E // nworkers // _K
    src = edge_index[0].reshape(nworkers, nch, _K)
    dst = edge_index[1].reshape(nworkers, nch, _K)
    W3p = jnp.pad(W3, ((0, 0), (0, F3 - C)))
    b3p = jnp.pad(b3, (0, F3 - C))

    ones16 = jnp.ones((_K, 16), jnp.float32)
    zeros16 = jnp.zeros((N, 16), jnp.float32)
    zerosH = jnp.zeros((N, H), jnp.float32)
    zerosF3 = jnp.zeros((N, F3), jnp.float32)

    deg_parts = _make_degree_kernel(N, E)(dst, ones16, zeros16)
    y1, dinv = _tc_first(x, W1, deg_parts)
    zp1 = _make_scatter_kernel(N, E, H)(y1, src, dst, zerosH)
    y2 = _tc_mid(zp1, y1, dinv, b1.reshape(1, H), W2)
    zp2 = _make_scatter_kernel(N, E, H)(y2, src, dst, zerosH)
    y3 = _tc_mid(zp2, y2, dinv, b2.reshape(1, H), W3p)
    zp3 = _make_scatter_kernel(N, E, F3)(y3, src, dst, zerosF3)
    out = _tc_last(zp3, y3, dinv, b3p.reshape(1, F3))
    return out[:, :C]


# speedup vs baseline: 1.2142x; 1.0064x over previous
import functools

import jax
import jax.numpy as jnp
from jax import lax
from jax.experimental import pallas as pl
from jax.experimental.pallas import tpu as pltpu
from jax.experimental.pallas import tpu_sc as plsc

_NC = 2
_NS = 16
_K = 100


def _sc_mesh():
    return plsc.VectorSubcoreMesh(
        core_axis_name="c", subcore_axis_name="s",
        num_cores=_NC, num_subcores=_NS)


def _make_degree_kernel(N, E):
    nworkers = _NC * _NS
    eper = E // nworkers
    nch = eper // _K
    rows = N // _NS

    @functools.partial(
        pl.kernel,
        out_type=jax.ShapeDtypeStruct((_NC, N, 16), jnp.float32),
        mesh=_sc_mesh(),
        compiler_params=pltpu.CompilerParams(use_tc_tiling_on_sc=False),
        scratch_types=[
            pltpu.VMEM((nch, _K), jnp.int32),
            pltpu.VMEM((_K, 16), jnp.float32),
            pltpu.VMEM_SHARED((N, 16), jnp.float32),
            pltpu.SemaphoreType.DMA,
            pltpu.SemaphoreType.DMA,
        ],
    )
    def deg_kernel(dst_hbm, ones_hbm, zeros_hbm, out_hbm, dst_v, ones_v,
                   acc_sh, sem_a, sem_b):
        c = lax.axis_index("c")
        s = lax.axis_index("s")
        wid = c * _NS + s
        pltpu.sync_copy(dst_hbm.at[wid], dst_v)
        pltpu.sync_copy(ones_hbm, ones_v)
        pltpu.sync_copy(zeros_hbm.at[pl.ds(s * rows, rows)],
                        acc_sh.at[pl.ds(s * rows, rows)])
        plsc.subcore_barrier()

        def scatter(i, sem):
            pltpu.async_copy(ones_v, acc_sh.at[dst_v.at[i]], sem, add=True)

        def swait(i, sem):
            pltpu.make_async_copy(ones_v, acc_sh.at[dst_v.at[i]], sem).wait()

        scatter(0, sem_a)

        def body(j, carry):
            i0 = 2 * j
            i1 = 2 * j + 1
            scatter(i1, sem_b)
            swait(i0, sem_a)

            @pl.when(i0 + 2 < nch)
            def _():
                scatter(i0 + 2, sem_a)

            swait(i1, sem_b)
            return carry

        lax.fori_loop(0, nch // 2, body, 0)
        plsc.subcore_barrier()
        pltpu.sync_copy(acc_sh.at[pl.ds(s * rows, rows)],
                        out_hbm.at[c].at[pl.ds(s * rows, rows)])

    return deg_kernel


def _make_scatter_kernel(N, E, F):
    nworkers = _NC * _NS
    eper = E // nworkers
    nch = eper // _K
    rows = N // _NS

    @functools.partial(
        pl.kernel,
        out_type=jax.ShapeDtypeStruct((_NC, N, F), jnp.float32),
        mesh=_sc_mesh(),
        compiler_params=pltpu.CompilerParams(use_tc_tiling_on_sc=False),
        scratch_types=[
            pltpu.VMEM((nch, _K), jnp.int32),
            pltpu.VMEM((nch, _K), jnp.int32),
            pltpu.VMEM((_K, F), jnp.float32),
            pltpu.VMEM((_K, F), jnp.float32),
            pltpu.VMEM_SHARED((N, F), jnp.float32),
            pltpu.SemaphoreType.DMA,
            pltpu.SemaphoreType.DMA,
        ],
    )
    def scatter_kernel(y_hbm, src_hbm, dst_hbm, zeros_hbm, out_hbm,
                       src_v, dst_v, rows_a, rows_b, acc_sh, sem_a, sem_b):
        c = lax.axis_index("c")
        s = lax.axis_index("s")
        wid = c * _NS + s
        pltpu.sync_copy(src_hbm.at[wid], src_v)
        pltpu.sync_copy(dst_hbm.at[wid], dst_v)

        def gather(i, buf, sem):
            return pltpu.async_copy(y_hbm.at[src_v.at[i]], buf, sem)

        def gwait(i, buf, sem):
            pltpu.make_async_copy(y_hbm.at[src_v.at[i]], buf, sem).wait()

        def scatter(i, buf):
            pltpu.sync_copy(buf, acc_sh.at[dst_v.at[i]], add=True)

        gather(0, rows_a, sem_a)
        pltpu.sync_copy(zeros_hbm.at[pl.ds(s * rows, rows)],
                        acc_sh.at[pl.ds(s * rows, rows)])
        plsc.subcore_barrier()


        def body(j, carry):
            i0 = 2 * j
            i1 = 2 * j + 1
            gather(i1, rows_b, sem_b)
            gwait(i0, rows_a, sem_a)
            scatter(i0, rows_a)

            @pl.when(i0 + 2 < nch)
            def _():
                gather(i0 + 2, rows_a, sem_a)

            gwait(i1, rows_b, sem_b)
            scatter(i1, rows_b)
            return carry

        lax.fori_loop(0, nch // 2, body, 0)
        plsc.subcore_barrier()
        pltpu.sync_copy(acc_sh.at[pl.ds(s * rows, rows)],
                        out_hbm.at[c].at[pl.ds(s * rows, rows)])

    return scatter_kernel


def _tc_first(x, W, deg_parts):
    N = x.shape[0]
    H = W.shape[1]

    def body(x_ref, w_ref, dp_ref, y_ref, dinv_ref):
        deg = dp_ref[0, :, 0:1] + dp_ref[1, :, 0:1] + 1.0
        dinv = lax.rsqrt(deg)
        xw = jnp.dot(x_ref[...], w_ref[...], preferred_element_type=jnp.float32)
        y_ref[...] = dinv * xw
        dinv_ref[...] = dinv

    return pl.pallas_call(
        body,
        out_shape=[jax.ShapeDtypeStruct((N, H), jnp.float32),
                   jax.ShapeDtypeStruct((N, 1), jnp.float32)],
    )(x, W, deg_parts)


def _tc_mid(zp, y, dinv, b, W):
    N = y.shape[0]
    H2 = W.shape[1]

    def body(zp_ref, y_ref, dinv_ref, b_ref, w_ref, out_ref):
        z = zp_ref[0] + zp_ref[1] + y_ref[...]
        h = jnp.maximum(dinv_ref[...] * z + b_ref[...], 0.0)
        out_ref[...] = dinv_ref[...] * jnp.dot(
            h, w_ref[...], preferred_element_type=jnp.float32)

    return pl.pallas_call(
        body,
        out_shape=jax.ShapeDtypeStruct((N, H2), jnp.float32),
    )(zp, y, dinv, b, W)


def _tc_last(zp, y, dinv, b):
    N, F = y.shape

    def body(zp_ref, y_ref, dinv_ref, b_ref, out_ref):
        z = zp_ref[0] + zp_ref[1] + y_ref[...]
        out_ref[...] = dinv_ref[...] * z + b_ref[...]

    return pl.pallas_call(
        body,
        out_shape=jax.ShapeDtypeStruct((N, F), jnp.float32),
    )(zp, y, dinv, b)


def kernel(x, edge_index, W1, b1, W2, b2, W3, b3):
    N = x.shape[0]
    E = edge_index.shape[1]
    H = W1.shape[1]
    C = W3.shape[1]
    F3 = 48

    nworkers = _NC * _NS
    nch = E // nworkers // _K
    src = edge_index[0].reshape(nworkers, nch, _K)
    dst = edge_index[1].reshape(nworkers, nch, _K)
    W3p = jnp.pad(W3, ((0, 0), (0, F3 - C)))
    b3p = jnp.pad(b3, (0, F3 - C))

    ones16 = jnp.ones((_K, 16), jnp.float32)
    zeros16 = jnp.zeros((N, 16), jnp.float32)
    zerosH = jnp.zeros((N, H), jnp.float32)
    zerosF3 = jnp.zeros((N, F3), jnp.float32)

    deg_parts = _make_degree_kernel(N, E)(dst, ones16, zeros16)
    y1, dinv = _tc_first(x, W1, deg_parts)
    zp1 = _make_scatter_kernel(N, E, H)(y1, src, dst, zerosH)
    y2 = _tc_mid(zp1, y1, dinv, b1.reshape(1, H), W2)
    zp2 = _make_scatter_kernel(N, E, H)(y2, src, dst, zerosH)
    y3 = _tc_mid(zp2, y2, dinv, b2.reshape(1, H), W3p)
    zp3 = _make_scatter_kernel(N, E, F3)(y3, src, dst, zerosF3)
    out = _tc_last(zp3, y3, dinv, b3p.reshape(1, F3))
    return out[:, :C]
